# baseline (speedup 1.0000x reference)
import jax
import jax.numpy as jnp
import numpy as np
from jax import lax
from jax.experimental import pallas as pl
from jax.experimental.pallas import tpu as pltpu
from jax.experimental.pallas import tpu_sc as plsc

_N_LEVELS = 16
_NF = 2
_LOG2_T = 19
_T = 2 ** _LOG2_T
_BASE_RES = 16
_FINEST_RES = 1024
_B_PTS = 1048576
_GROWTH = np.float32(
    np.exp((np.log(np.float32(_FINEST_RES)) - np.log(np.float32(_BASE_RES)))
           / (_N_LEVELS - 1)))
_PRIME1_I32 = int(np.uint32(2654435761).view(np.int32))
_HASH_MASK = _T - 1

_RES = []
_RESF = []
_GS = []
_TSIZE = []
_DENSE = []
for _i in range(_N_LEVELS):
    _resf = float(np.floor(np.float32(_BASE_RES) * _GROWTH ** np.float32(_i)))
    _r = int(_resf)
    _RES.append(_r)
    _RESF.append(np.float32(_resf))
    _GS.append(np.float32(1.0 / _resf))
    if _r * _r < _T:
        _TSIZE.append((_r + 1) ** 2)
        _DENSE.append(True)
    else:
        _TSIZE.append(_T)
        _DENSE.append(False)

_NC = 2
_NS = 16
_NW = _NC * _NS
_PW = _B_PTS // _NW
_C = 128
_VPC = _C // 16
_NCHUNK = _PW // _C

_RES_LEVELS = list(range(0, 8))
_BIG_LEVELS = list(range(8, 16))
_NGATH = [1 if _DENSE[l] else 4 for l in _BIG_LEVELS]
_GW = 8
_GOFF = list(np.cumsum([0] + _NGATH))
_TOTG = _GOFF[-1]


def _coords(x0, x1, l):
    rf = _RESF[l]
    b0 = (x0 * rf).astype(jnp.int32)
    b1 = (x1 * rf).astype(jnp.int32)
    return b0, b1


def _weights(x0, x1, b0, b1, l):
    gs = _GS[l]
    rf = _RESF[l]
    w0 = (x0 - b0.astype(jnp.float32) * gs) * rf
    w1 = (x1 - b1.astype(jnp.float32) * gs) * rf
    return w0, w1


def _corner_indices(b0, b1, l):
    if _DENSE[l]:
        r = _RES[l]
        i00 = b0 * r + b1
        return i00, i00 + 1, i00 + r, i00 + r + 1
    h0 = b0 ^ (b1 * _PRIME1_I32)
    h1 = b0 ^ ((b1 + 1) * _PRIME1_I32)
    h2 = (b0 + 1) ^ (b1 * _PRIME1_I32)
    h3 = (b0 + 1) ^ ((b1 + 1) * _PRIME1_I32)
    return (h0 & _HASH_MASK, h1 & _HASH_MASK,
            h2 & _HASH_MASK, h3 & _HASH_MASK)


def _lerp_store(corner_vals, w0, w1, out_ref, prow, l):
    u0 = 1.0 - w0
    u1 = 1.0 - w1
    for f in range(_NF):
        e00, e01, e10, e11 = corner_vals[f]
        c0 = e00 * u1 + e01 * w1
        c1 = e10 * u1 + e11 * w1
        o = c0 * u0 + c1 * w0
        plsc.store_scatter(
            out_ref, [jnp.full((16,), 2 * l + f, jnp.int32), prow], o)


def _hash_rows(i):
    p0 = ((i >> 7) << 8) + (i & 127)
    return p0 >> 3


def _sc_body(x_hbm, *refs):
    tbl_hbm = refs[0:_N_LEVELS]
    out_hbm = refs[_N_LEVELS]
    h_hbm = refs[_N_LEVELS + 1:_N_LEVELS + 3]
    s = refs[_N_LEVELS + 3:]
    nres = len(_RES_LEVELS)
    tbl_v = s[0:nres]
    p = nres
    xi_v = s[p:p + 2]; p += 2
    x0_v = s[p:p + 2]; p += 2
    x1_v = s[p:p + 2]; p += 2
    out_v = s[p]; p += 1
    idx_v = [s[p:p + _TOTG], s[p + _TOTG:p + 2 * _TOTG]]; p += 2 * _TOTG
    rows_v = [s[p:p + _TOTG], s[p + _TOTG:p + 2 * _TOTG]]; p += 2 * _TOTG
    gsem = s[p:p + 2]

    wid = lax.axis_index("s") * _NC + lax.axis_index("c")
    sid = lax.axis_index("s")
    cbase = wid * _NCHUNK

    lane = lax.iota(jnp.int32, 16)

    pk_stage = rows_v[0][0:8]

    def pack_body(k, c):
        for hl in range(2):
            src = tbl_hbm[14 + hl]
            pltpu.sync_copy(src.at[pl.ds(32 * k, 32), :],
                            pk_stage[2 * hl].at[pl.ds(0, 32), :])
            st = pk_stage[2 * hl]
            dst = pk_stage[2 * hl + 1]

            def pv(v, cc):
                j = v * 16 + lane
                z = jnp.zeros((16,), jnp.int32)
                f0 = plsc.load_gather(st, [(j >> 3), j & 7])
                jj = j + 128
                f1 = plsc.load_gather(st, [(jj >> 3), jj & 7])
                plsc.store_scatter(dst, [j, z], f0)
                plsc.store_scatter(dst, [j, z + 1], f1)
                return cc

            lax.fori_loop(0, _VPC, pv, 0)
            pltpu.sync_copy(dst, h_hbm[hl].at[pl.ds(128 * k, 128), :])
        return c

    def pack_loop(i, c):
        return pack_body(sid + i * _NS, c)
    lax.fori_loop(0, _T // _C // _NS, pack_loop, 0)
    plsc.subcore_barrier()

    for li, l in enumerate(_RES_LEVELS):
        pltpu.sync_copy(tbl_hbm[l], tbl_v[li])

    def fire(par, g):
        pltpu.sync_copy(x_hbm.at[g], xi_v[par])

        def deint_body(v, c):
            x0_v[par][pl.ds(v * 16, 16)] = xi_v[par][pl.ds(v * 16, 16)]
            x1_v[par][pl.ds(v * 16, 16)] = xi_v[par][pl.ds(128 + v * 16, 16)]
            return c

        lax.fori_loop(0, _VPC, deint_body, 0)

        def idx_body(v, c):
            x0 = x0_v[par][pl.ds(v * 16, 16)]
            x1 = x1_v[par][pl.ds(v * 16, 16)]
            for li, l in enumerate(_BIG_LEVELS):
                b0, b1 = _coords(x0, x1, l)
                i00, i01, i10, i11 = _corner_indices(b0, b1, l)
                if _DENSE[l]:
                    ivs = (i00,)
                else:
                    ivs = (i00, i01, i10, i11)
                for ci, iv in enumerate(ivs):
                    idx_v[par][_GOFF[li] + ci][pl.ds(v * 16, 16)] = iv
            return c

        lax.fori_loop(0, _VPC, idx_body, 0)

        for li, l in enumerate(_BIG_LEVELS):
            src = tbl_hbm[l] if _DENSE[l] else h_hbm[l - 14]
            for ci in range(_NGATH[li]):
                gi = _GOFF[li] + ci
                pltpu.async_copy(
                    src.at[idx_v[par][gi]], rows_v[par][gi], gsem[par])

    def wait_gathers(par):
        for li, l in enumerate(_BIG_LEVELS):
            src = tbl_hbm[l] if _DENSE[l] else h_hbm[l - 14]
            for ci in range(_NGATH[li]):
                gi = _GOFF[li] + ci
                pltpu.make_async_copy(
                    src.at[idx_v[par][gi]], rows_v[par][gi],
                    gsem[par]).wait()

    def combine(par, g):
        def res_body(v, c):
            x0 = x0_v[par][pl.ds(v * 16, 16)]
            x1 = x1_v[par][pl.ds(v * 16, 16)]
            prow = v * 16 + lane
            for li, l in enumerate(_RES_LEVELS):
                b0, b1 = _coords(x0, x1, l)
                w0, w1 = _weights(x0, x1, b0, b1, l)
                i00, i01, i10, i11 = _corner_indices(b0, b1, l)
                vals = []
                for f in range(_NF):
                    vals.append(tuple(
                        plsc.load_gather(tbl_v[li], [2 * i + f])
                        for i in (i00, i01, i10, i11)))
                _lerp_store(vals, w0, w1, out_v, prow, l)
            return c

        lax.fori_loop(0, _VPC, res_body, 0)

        wait_gathers(par)

        def big_body(v, c):
            x0 = x0_v[par][pl.ds(v * 16, 16)]
            x1 = x1_v[par][pl.ds(v * 16, 16)]
            prow = v * 16 + lane
            lidx = prow
            for li, l in enumerate(_BIG_LEVELS):
                b0, b1 = _coords(x0, x1, l)
                w0, w1 = _weights(x0, x1, b0, b1, l)
                gi = _GOFF[li]
                vals = []
                if _DENSE[l]:
                    for f in range(_NF):
                        rv = rows_v[par]
                        qcols = [jnp.full((16,), 2 * k + f, jnp.int32)
                                 for k in range(4)]
                        e00 = plsc.load_gather(rv[gi], [lidx, qcols[0]])
                        e01 = plsc.load_gather(rv[gi], [lidx, qcols[1]])
                        e10 = plsc.load_gather(rv[gi], [lidx, qcols[2]])
                        e11 = plsc.load_gather(rv[gi], [lidx, qcols[3]])
                        vals.append((e00, e01, e10, e11))
                else:
                    for f in range(_NF):
                        rv = rows_v[par]
                        col = jnp.full((16,), f, jnp.int32)
                        e00 = plsc.load_gather(rv[gi + 0], [lidx, col])
                        e01 = plsc.load_gather(rv[gi + 1], [lidx, col])
                        e10 = plsc.load_gather(rv[gi + 2], [lidx, col])
                        e11 = plsc.load_gather(rv[gi + 3], [lidx, col])
                        vals.append((e00, e01, e10, e11))
                _lerp_store(vals, w0, w1, out_v, prow, l)
            return c

        lax.fori_loop(0, _VPC, big_body, 0)

        for a in range(4):
            pltpu.sync_copy(out_v.at[pl.ds(8 * a, 8), :], out_hbm.at[a, g])

    fire(0, cbase)

    def body(i, carry):
        g0 = cbase + 2 * i
        fire(1, g0 + 1)
        combine(0, g0)

        @pl.when(i < _NCHUNK // 2 - 1)
        def _fire_next():
            fire(0, g0 + 2)

        combine(1, g0 + 1)
        return carry

    lax.fori_loop(0, _NCHUNK // 2, body, 0)


def _make_kernel():
    scratch = []
    scratch += [pltpu.VMEM((_TSIZE[l] * _NF,), jnp.float32)
                for l in _RES_LEVELS]
    scratch += [pltpu.VMEM((2 * _C,), jnp.float32)] * 2
    scratch += [pltpu.VMEM((_C,), jnp.float32)] * 2
    scratch += [pltpu.VMEM((_C,), jnp.float32)] * 2
    scratch += [pltpu.VMEM((2 * _N_LEVELS, _C), jnp.float32)]
    scratch += [pltpu.VMEM((_C,), jnp.int32) for _ in range(2 * _TOTG)]
    scratch += [pltpu.VMEM((_C, _GW), jnp.float32) for _ in range(2 * _TOTG)]
    scratch += [pltpu.SemaphoreType.DMA] * 2
    mesh = plsc.VectorSubcoreMesh(core_axis_name="c", subcore_axis_name="s")
    return pl.kernel(
        _sc_body,
        out_type=(jax.ShapeDtypeStruct((4, _B_PTS // _C, 8, _C), jnp.float32),
                  jax.ShapeDtypeStruct((_T, 8), jnp.float32),
                  jax.ShapeDtypeStruct((_T, 8), jnp.float32)),
        mesh=mesh,
        scratch_types=scratch,
        compiler_params=pltpu.CompilerParams(
            needs_layout_passes=False, use_tc_tiling_on_sc=False),
    )


_sc_kernel = _make_kernel()


@jax.jit
def kernel(x, emb_0, emb_1, emb_2, emb_3, emb_4, emb_5, emb_6, emb_7,
           emb_8, emb_9, emb_10, emb_11, emb_12, emb_13, emb_14, emb_15):
    tables = [emb_0, emb_1, emb_2, emb_3, emb_4, emb_5, emb_6, emb_7,
              emb_8, emb_9, emb_10, emb_11, emb_12, emb_13, emb_14, emb_15]
    args = []
    for l in range(_N_LEVELS):
        tb = tables[l]
        if l in _RES_LEVELS:
            args.append(tb.reshape(-1))
        elif _DENSE[l]:
            r = _RES[l]
            args.append(jnp.concatenate(
                [tb, jnp.roll(tb, -1, axis=0),
                 jnp.roll(tb, -r, axis=0), jnp.roll(tb, -r - 1, axis=0)],
                axis=1))
        else:
            args.append(tb.reshape(_T // _C, _C, _NF)
                        .transpose(0, 2, 1).reshape(_T // 4, 8))
    xv = x.reshape(_B_PTS // _C, _C, 2).transpose(0, 2, 1).reshape(
        _B_PTS // _C, 2 * _C)
    out4, _h14, _h15 = _sc_kernel(xv, *args)
    return out4.transpose(1, 3, 0, 2).reshape(_B_PTS, 2 * _N_LEVELS)


# speedup vs baseline: 1.1877x; 1.1877x over previous
import jax
import jax.numpy as jnp
import numpy as np
from jax import lax
from jax.experimental import pallas as pl
from jax.experimental.pallas import tpu as pltpu
from jax.experimental.pallas import tpu_sc as plsc

_N_LEVELS = 16
_NF = 2
_LOG2_T = 19
_T = 2 ** _LOG2_T
_BASE_RES = 16
_FINEST_RES = 1024
_B_PTS = 1048576
_GROWTH = np.float32(
    np.exp((np.log(np.float32(_FINEST_RES)) - np.log(np.float32(_BASE_RES)))
           / (_N_LEVELS - 1)))
_PRIME1_I32 = int(np.uint32(2654435761).view(np.int32))
_HASH_MASK = _T - 1

_RES = []
_RESF = []
_GS = []
_TSIZE = []
_DENSE = []
for _i in range(_N_LEVELS):
    _resf = float(np.floor(np.float32(_BASE_RES) * _GROWTH ** np.float32(_i)))
    _r = int(_resf)
    _RES.append(_r)
    _RESF.append(np.float32(_resf))
    _GS.append(np.float32(1.0 / _resf))
    if _r * _r < _T:
        _TSIZE.append((_r + 1) ** 2)
        _DENSE.append(True)
    else:
        _TSIZE.append(_T)
        _DENSE.append(False)

_NC = 2
_NS = 16
_NW = _NC * _NS
_PW = _B_PTS // _NW
_C = 128
_VPC = _C // 16
_NCHUNK = _PW // _C

_RES_LEVELS = list(range(0, 8))
_BIG_LEVELS = list(range(8, 16))
_NGATH = [1 if _DENSE[l] else 4 for l in _BIG_LEVELS]
_GW = 8
_GOFF = list(np.cumsum([0] + _NGATH))
_TOTG = _GOFF[-1]


def _coords(x0, x1, l):
    rf = _RESF[l]
    b0 = (x0 * rf).astype(jnp.int32)
    b1 = (x1 * rf).astype(jnp.int32)
    return b0, b1


def _weights(x0, x1, b0, b1, l):
    gs = _GS[l]
    rf = _RESF[l]
    w0 = (x0 - b0.astype(jnp.float32) * gs) * rf
    w1 = (x1 - b1.astype(jnp.float32) * gs) * rf
    return w0, w1


def _corner_indices(b0, b1, l):
    if _DENSE[l]:
        r = _RES[l]
        i00 = b0 * r + b1
        return i00, i00 + 1, i00 + r, i00 + r + 1
    h0 = b0 ^ (b1 * _PRIME1_I32)
    h1 = b0 ^ ((b1 + 1) * _PRIME1_I32)
    h2 = (b0 + 1) ^ (b1 * _PRIME1_I32)
    h3 = (b0 + 1) ^ ((b1 + 1) * _PRIME1_I32)
    return (h0 & _HASH_MASK, h1 & _HASH_MASK,
            h2 & _HASH_MASK, h3 & _HASH_MASK)


def _lerp_store(corner_vals, w0, w1, out_ref, prow, l):
    u0 = 1.0 - w0
    u1 = 1.0 - w1
    for f in range(_NF):
        e00, e01, e10, e11 = corner_vals[f]
        c0 = e00 * u1 + e01 * w1
        c1 = e10 * u1 + e11 * w1
        o = c0 * u0 + c1 * w0
        plsc.store_scatter(
            out_ref, [jnp.full((16,), 2 * l + f, jnp.int32), prow], o)


def _hash_rows(i):
    p0 = ((i >> 7) << 8) + (i & 127)
    return p0 >> 3


def _sc_body(x_hbm, *refs):
    tbl_hbm = refs[0:_N_LEVELS]
    out_hbm = refs[_N_LEVELS]
    h_hbm = refs[_N_LEVELS + 1:_N_LEVELS + 3]
    s = refs[_N_LEVELS + 3:]
    nres = len(_RES_LEVELS)
    tbl_v = s[0:nres]
    p = nres
    xi_v = s[p:p + 2]; p += 2
    x0_v = s[p:p + 2]; p += 2
    x1_v = s[p:p + 2]; p += 2
    out_v = s[p]; p += 1
    idx_v = [s[p:p + _TOTG], s[p + _TOTG:p + 2 * _TOTG]]; p += 2 * _TOTG
    rows_v = [s[p:p + _TOTG], s[p + _TOTG:p + 2 * _TOTG]]; p += 2 * _TOTG
    gsem = s[p:p + 2]; p += 2
    pack_in, pack_out = s[p], s[p + 1]

    wid = lax.axis_index("s") * _NC + lax.axis_index("c")
    sid = lax.axis_index("s")
    cbase = wid * _NCHUNK

    lane = lax.iota(jnp.int32, 16)

    _PB = 2048
    _PBLK = _T // _PB // _NS

    def pack_body(i, c):
        for hl in range(2):
            k = sid + i * _NS
            src = tbl_hbm[14 + hl]
            pltpu.sync_copy(src.at[pl.ds((_PB // 4) * k, _PB // 4), :],
                            pack_in)
            def pv(v, cc):
                cl = v * 16 + lane
                z = jnp.zeros((16,), jnp.int32)
                j0 = ((cl >> 7) << 8) + (cl & 127)
                f0 = plsc.load_gather(pack_in, [(j0 >> 3), j0 & 7])
                j1 = j0 + 128
                f1 = plsc.load_gather(pack_in, [(j1 >> 3), j1 & 7])
                plsc.store_scatter(pack_out, [cl, z], f0)
                plsc.store_scatter(pack_out, [cl, z + 1], f1)
                return cc

            lax.fori_loop(0, _PB // 16, pv, 0)
            pltpu.sync_copy(pack_out, h_hbm[hl].at[pl.ds(_PB * k, _PB), :])
        return c

    lax.fori_loop(0, _PBLK, pack_body, 0)
    plsc.subcore_barrier()

    for li, l in enumerate(_RES_LEVELS):
        pltpu.sync_copy(tbl_hbm[l], tbl_v[li])

    def fire(par, g):
        pltpu.sync_copy(x_hbm.at[g], xi_v[par])

        def deint_body(v, c):
            x0_v[par][pl.ds(v * 16, 16)] = xi_v[par][pl.ds(v * 16, 16)]
            x1_v[par][pl.ds(v * 16, 16)] = xi_v[par][pl.ds(128 + v * 16, 16)]
            return c

        lax.fori_loop(0, _VPC, deint_body, 0)

        def idx_body(v, c):
            x0 = x0_v[par][pl.ds(v * 16, 16)]
            x1 = x1_v[par][pl.ds(v * 16, 16)]
            for li, l in enumerate(_BIG_LEVELS):
                b0, b1 = _coords(x0, x1, l)
                i00, i01, i10, i11 = _corner_indices(b0, b1, l)
                if _DENSE[l]:
                    ivs = (i00,)
                else:
                    ivs = (i00, i01, i10, i11)
                for ci, iv in enumerate(ivs):
                    idx_v[par][_GOFF[li] + ci][pl.ds(v * 16, 16)] = iv
            return c

        lax.fori_loop(0, _VPC, idx_body, 0)

        for li, l in enumerate(_BIG_LEVELS):
            src = tbl_hbm[l] if _DENSE[l] else h_hbm[l - 14]
            for ci in range(_NGATH[li]):
                gi = _GOFF[li] + ci
                pltpu.async_copy(
                    src.at[idx_v[par][gi]], rows_v[par][gi], gsem[par])

    def wait_gathers(par):
        for li, l in enumerate(_BIG_LEVELS):
            src = tbl_hbm[l] if _DENSE[l] else h_hbm[l - 14]
            for ci in range(_NGATH[li]):
                gi = _GOFF[li] + ci
                pltpu.make_async_copy(
                    src.at[idx_v[par][gi]], rows_v[par][gi],
                    gsem[par]).wait()

    def combine(par, g):
        def res_body(v, c):
            x0 = x0_v[par][pl.ds(v * 16, 16)]
            x1 = x1_v[par][pl.ds(v * 16, 16)]
            prow = v * 16 + lane
            for li, l in enumerate(_RES_LEVELS):
                b0, b1 = _coords(x0, x1, l)
                w0, w1 = _weights(x0, x1, b0, b1, l)
                i00, i01, i10, i11 = _corner_indices(b0, b1, l)
                vals = []
                for f in range(_NF):
                    vals.append(tuple(
                        plsc.load_gather(tbl_v[li], [2 * i + f])
                        for i in (i00, i01, i10, i11)))
                _lerp_store(vals, w0, w1, out_v, prow, l)
            return c

        lax.fori_loop(0, _VPC, res_body, 0)

        wait_gathers(par)

        def big_body(v, c):
            x0 = x0_v[par][pl.ds(v * 16, 16)]
            x1 = x1_v[par][pl.ds(v * 16, 16)]
            prow = v * 16 + lane
            lidx = prow
            for li, l in enumerate(_BIG_LEVELS):
                b0, b1 = _coords(x0, x1, l)
                w0, w1 = _weights(x0, x1, b0, b1, l)
                gi = _GOFF[li]
                vals = []
                if _DENSE[l]:
                    for f in range(_NF):
                        rv = rows_v[par]
                        qcols = [jnp.full((16,), 2 * k + f, jnp.int32)
                                 for k in range(4)]
                        e00 = plsc.load_gather(rv[gi], [lidx, qcols[0]])
                        e01 = plsc.load_gather(rv[gi], [lidx, qcols[1]])
                        e10 = plsc.load_gather(rv[gi], [lidx, qcols[2]])
                        e11 = plsc.load_gather(rv[gi], [lidx, qcols[3]])
                        vals.append((e00, e01, e10, e11))
                else:
                    for f in range(_NF):
                        rv = rows_v[par]
                        col = jnp.full((16,), f, jnp.int32)
                        e00 = plsc.load_gather(rv[gi + 0], [lidx, col])
                        e01 = plsc.load_gather(rv[gi + 1], [lidx, col])
                        e10 = plsc.load_gather(rv[gi + 2], [lidx, col])
                        e11 = plsc.load_gather(rv[gi + 3], [lidx, col])
                        vals.append((e00, e01, e10, e11))
                _lerp_store(vals, w0, w1, out_v, prow, l)
            return c

        lax.fori_loop(0, _VPC, big_body, 0)

        for a in range(4):
            pltpu.sync_copy(out_v.at[pl.ds(8 * a, 8), :], out_hbm.at[a, g])

    fire(0, cbase)

    def body(i, carry):
        g0 = cbase + 2 * i
        fire(1, g0 + 1)
        combine(0, g0)

        @pl.when(i < _NCHUNK // 2 - 1)
        def _fire_next():
            fire(0, g0 + 2)

        combine(1, g0 + 1)
        return carry

    lax.fori_loop(0, _NCHUNK // 2, body, 0)


def _make_kernel():
    scratch = []
    scratch += [pltpu.VMEM((_TSIZE[l] * _NF,), jnp.float32)
                for l in _RES_LEVELS]
    scratch += [pltpu.VMEM((2 * _C,), jnp.float32)] * 2
    scratch += [pltpu.VMEM((_C,), jnp.float32)] * 2
    scratch += [pltpu.VMEM((_C,), jnp.float32)] * 2
    scratch += [pltpu.VMEM((2 * _N_LEVELS, _C), jnp.float32)]
    scratch += [pltpu.VMEM((_C,), jnp.int32) for _ in range(2 * _TOTG)]
    scratch += [pltpu.VMEM((_C, _GW), jnp.float32) for _ in range(2 * _TOTG)]
    scratch += [pltpu.SemaphoreType.DMA] * 2
    scratch += [pltpu.VMEM((512, _GW), jnp.float32),
                pltpu.VMEM((2048, _GW), jnp.float32)]
    mesh = plsc.VectorSubcoreMesh(core_axis_name="c", subcore_axis_name="s")
    return pl.kernel(
        _sc_body,
        out_type=(jax.ShapeDtypeStruct((4, _B_PTS // _C, 8, _C), jnp.float32),
                  jax.ShapeDtypeStruct((_T, 8), jnp.float32),
                  jax.ShapeDtypeStruct((_T, 8), jnp.float32)),
        mesh=mesh,
        scratch_types=scratch,
        compiler_params=pltpu.CompilerParams(
            needs_layout_passes=False, use_tc_tiling_on_sc=False),
    )


_sc_kernel = _make_kernel()


@jax.jit
def kernel(x, emb_0, emb_1, emb_2, emb_3, emb_4, emb_5, emb_6, emb_7,
           emb_8, emb_9, emb_10, emb_11, emb_12, emb_13, emb_14, emb_15):
    tables = [emb_0, emb_1, emb_2, emb_3, emb_4, emb_5, emb_6, emb_7,
              emb_8, emb_9, emb_10, emb_11, emb_12, emb_13, emb_14, emb_15]
    args = []
    for l in range(_N_LEVELS):
        tb = tables[l]
        if l in _RES_LEVELS:
            args.append(tb.reshape(-1))
        elif _DENSE[l]:
            r = _RES[l]
            args.append(jnp.concatenate(
                [tb, jnp.roll(tb, -1, axis=0),
                 jnp.roll(tb, -r, axis=0), jnp.roll(tb, -r - 1, axis=0)],
                axis=1))
        else:
            args.append(tb.reshape(_T // _C, _C, _NF)
                        .transpose(0, 2, 1).reshape(_T // 4, 8))
    xv = x.reshape(_B_PTS // _C, _C, 2).transpose(0, 2, 1).reshape(
        _B_PTS // _C, 2 * _C)
    out4, _h14, _h15 = _sc_kernel(xv, *args)
    return out4.transpose(1, 3, 0, 2).reshape(_B_PTS, 2 * _N_LEVELS)


# speedup vs baseline: 1.1981x; 1.0087x over previous
import jax
import jax.numpy as jnp
import numpy as np
from jax import lax
from jax.experimental import pallas as pl
from jax.experimental.pallas import tpu as pltpu
from jax.experimental.pallas import tpu_sc as plsc

_N_LEVELS = 16
_NF = 2
_LOG2_T = 19
_T = 2 ** _LOG2_T
_BASE_RES = 16
_FINEST_RES = 1024
_B_PTS = 1048576
_GROWTH = np.float32(
    np.exp((np.log(np.float32(_FINEST_RES)) - np.log(np.float32(_BASE_RES)))
           / (_N_LEVELS - 1)))
_PRIME1_I32 = int(np.uint32(2654435761).view(np.int32))
_HASH_MASK = _T - 1

_RES = []
_RESF = []
_GS = []
_TSIZE = []
_DENSE = []
for _i in range(_N_LEVELS):
    _resf = float(np.floor(np.float32(_BASE_RES) * _GROWTH ** np.float32(_i)))
    _r = int(_resf)
    _RES.append(_r)
    _RESF.append(np.float32(_resf))
    _GS.append(np.float32(1.0 / _resf))
    if _r * _r < _T:
        _TSIZE.append((_r + 1) ** 2)
        _DENSE.append(True)
    else:
        _TSIZE.append(_T)
        _DENSE.append(False)

_NC = 2
_NS = 16
_NW = _NC * _NS
_PW = _B_PTS // _NW
_C = 128
_VPC = _C // 16
_NCHUNK = _PW // _C

_RES_LEVELS = list(range(0, 8))
_BIG_LEVELS = list(range(8, 16))
_NGATH = [1 if _DENSE[l] else 8 for l in _BIG_LEVELS]
_GW = 8
_GOFF = list(np.cumsum([0] + _NGATH))
_TOTG = _GOFF[-1]


def _coords(x0, x1, l):
    rf = _RESF[l]
    b0 = (x0 * rf).astype(jnp.int32)
    b1 = (x1 * rf).astype(jnp.int32)
    return b0, b1


def _weights(x0, x1, b0, b1, l):
    gs = _GS[l]
    rf = _RESF[l]
    w0 = (x0 - b0.astype(jnp.float32) * gs) * rf
    w1 = (x1 - b1.astype(jnp.float32) * gs) * rf
    return w0, w1


def _corner_indices(b0, b1, l):
    if _DENSE[l]:
        r = _RES[l]
        i00 = b0 * r + b1
        return i00, i00 + 1, i00 + r, i00 + r + 1
    h0 = b0 ^ (b1 * _PRIME1_I32)
    h1 = b0 ^ ((b1 + 1) * _PRIME1_I32)
    h2 = (b0 + 1) ^ (b1 * _PRIME1_I32)
    h3 = (b0 + 1) ^ ((b1 + 1) * _PRIME1_I32)
    return (h0 & _HASH_MASK, h1 & _HASH_MASK,
            h2 & _HASH_MASK, h3 & _HASH_MASK)


def _lerp_store(corner_vals, w0, w1, out_ref, prow, l):
    u0 = 1.0 - w0
    u1 = 1.0 - w1
    for f in range(_NF):
        e00, e01, e10, e11 = corner_vals[f]
        c0 = e00 * u1 + e01 * w1
        c1 = e10 * u1 + e11 * w1
        o = c0 * u0 + c1 * w0
        plsc.store_scatter(
            out_ref, [jnp.full((16,), 2 * l + f, jnp.int32), prow], o)


def _hash_rows(i):
    p0 = ((i >> 7) << 8) + (i & 127)
    return p0 >> 3


def _sc_body(x_hbm, *refs):
    tbl_hbm = refs[0:_N_LEVELS]
    out_hbm = refs[_N_LEVELS]
    s = refs[_N_LEVELS + 1:]
    nres = len(_RES_LEVELS)
    tbl_v = s[0:nres]
    p = nres
    xi_v = s[p:p + 2]; p += 2
    x0_v = s[p:p + 2]; p += 2
    x1_v = s[p:p + 2]; p += 2
    out_v = s[p]; p += 1
    idx_v = [s[p:p + _TOTG], s[p + _TOTG:p + 2 * _TOTG]]; p += 2 * _TOTG
    rows_v = [s[p:p + _TOTG], s[p + _TOTG:p + 2 * _TOTG]]; p += 2 * _TOTG
    gsem = s[p:p + 2]

    wid = lax.axis_index("s") * _NC + lax.axis_index("c")
    cbase = wid * _NCHUNK

    for li, l in enumerate(_RES_LEVELS):
        pltpu.sync_copy(tbl_hbm[l], tbl_v[li])

    lane = lax.iota(jnp.int32, 16)

    def fire(par, g):
        pltpu.sync_copy(x_hbm.at[g], xi_v[par])

        def deint_body(v, c):
            x0_v[par][pl.ds(v * 16, 16)] = xi_v[par][pl.ds(v * 16, 16)]
            x1_v[par][pl.ds(v * 16, 16)] = xi_v[par][pl.ds(128 + v * 16, 16)]
            return c

        lax.fori_loop(0, _VPC, deint_body, 0)

        def idx_body(v, c):
            x0 = x0_v[par][pl.ds(v * 16, 16)]
            x1 = x1_v[par][pl.ds(v * 16, 16)]
            for li, l in enumerate(_BIG_LEVELS):
                b0, b1 = _coords(x0, x1, l)
                i00, i01, i10, i11 = _corner_indices(b0, b1, l)
                if _DENSE[l]:
                    ivs = (i00,)
                else:
                    r00, r01 = _hash_rows(i00), _hash_rows(i01)
                    r10, r11 = _hash_rows(i10), _hash_rows(i11)
                    ivs = (r00, r01, r10, r11,
                           r00 + 16, r01 + 16, r10 + 16, r11 + 16)
                for ci, iv in enumerate(ivs):
                    idx_v[par][_GOFF[li] + ci][pl.ds(v * 16, 16)] = iv
            return c

        lax.fori_loop(0, _VPC, idx_body, 0)

        for li, l in enumerate(_BIG_LEVELS):
            for ci in range(_NGATH[li]):
                gi = _GOFF[li] + ci
                pltpu.async_copy(
                    tbl_hbm[l].at[idx_v[par][gi]], rows_v[par][gi],
                    gsem[par])

    def wait_gathers(par):
        for li, l in enumerate(_BIG_LEVELS):
            for ci in range(_NGATH[li]):
                gi = _GOFF[li] + ci
                pltpu.make_async_copy(
                    tbl_hbm[l].at[idx_v[par][gi]], rows_v[par][gi],
                    gsem[par]).wait()

    def combine(par, g):
        def res_body(v, c):
            x0 = x0_v[par][pl.ds(v * 16, 16)]
            x1 = x1_v[par][pl.ds(v * 16, 16)]
            prow = v * 16 + lane
            for li, l in enumerate(_RES_LEVELS):
                b0, b1 = _coords(x0, x1, l)
                w0, w1 = _weights(x0, x1, b0, b1, l)
                i00, i01, i10, i11 = _corner_indices(b0, b1, l)
                vals = []
                for f in range(_NF):
                    vals.append(tuple(
                        plsc.load_gather(tbl_v[li], [2 * i + f])
                        for i in (i00, i01, i10, i11)))
                _lerp_store(vals, w0, w1, out_v, prow, l)
            return c

        lax.fori_loop(0, _VPC, res_body, 0)

        wait_gathers(par)

        def big_body(v, c):
            x0 = x0_v[par][pl.ds(v * 16, 16)]
            x1 = x1_v[par][pl.ds(v * 16, 16)]
            prow = v * 16 + lane
            lidx = prow
            for li, l in enumerate(_BIG_LEVELS):
                b0, b1 = _coords(x0, x1, l)
                w0, w1 = _weights(x0, x1, b0, b1, l)
                gi = _GOFF[li]
                vals = []
                if _DENSE[l]:
                    for f in range(_NF):
                        rv = rows_v[par]
                        qcols = [jnp.full((16,), 2 * k + f, jnp.int32)
                                 for k in range(4)]
                        e00 = plsc.load_gather(rv[gi], [lidx, qcols[0]])
                        e01 = plsc.load_gather(rv[gi], [lidx, qcols[1]])
                        e10 = plsc.load_gather(rv[gi], [lidx, qcols[2]])
                        e11 = plsc.load_gather(rv[gi], [lidx, qcols[3]])
                        vals.append((e00, e01, e10, e11))
                else:
                    i00, i01, i10, i11 = _corner_indices(b0, b1, l)
                    for f in range(_NF):
                        o = 4 * f
                        rv = rows_v[par]
                        e00 = plsc.load_gather(rv[gi + o + 0], [lidx, i00 & 7])
                        e01 = plsc.load_gather(rv[gi + o + 1], [lidx, i01 & 7])
                        e10 = plsc.load_gather(rv[gi + o + 2], [lidx, i10 & 7])
                        e11 = plsc.load_gather(rv[gi + o + 3], [lidx, i11 & 7])
                        vals.append((e00, e01, e10, e11))
                _lerp_store(vals, w0, w1, out_v, prow, l)
            return c

        lax.fori_loop(0, _VPC, big_body, 0)

        for a in range(4):
            pltpu.sync_copy(out_v.at[pl.ds(8 * a, 8), :], out_hbm.at[a, g])

    fire(0, cbase)

    def body(i, carry):
        g0 = cbase + 2 * i
        fire(1, g0 + 1)
        combine(0, g0)

        @pl.when(i < _NCHUNK // 2 - 1)
        def _fire_next():
            fire(0, g0 + 2)

        combine(1, g0 + 1)
        return carry

    lax.fori_loop(0, _NCHUNK // 2, body, 0)


def _make_kernel():
    scratch = []
    scratch += [pltpu.VMEM((_TSIZE[l] * _NF,), jnp.float32)
                for l in _RES_LEVELS]
    scratch += [pltpu.VMEM((2 * _C,), jnp.float32)] * 2
    scratch += [pltpu.VMEM((_C,), jnp.float32)] * 2
    scratch += [pltpu.VMEM((_C,), jnp.float32)] * 2
    scratch += [pltpu.VMEM((2 * _N_LEVELS, _C), jnp.float32)]
    scratch += [pltpu.VMEM((_C,), jnp.int32) for _ in range(2 * _TOTG)]
    scratch += [pltpu.VMEM((_C, _GW), jnp.float32) for _ in range(2 * _TOTG)]
    scratch += [pltpu.SemaphoreType.DMA] * 2
    mesh = plsc.VectorSubcoreMesh(core_axis_name="c", subcore_axis_name="s")
    return pl.kernel(
        _sc_body,
        out_type=jax.ShapeDtypeStruct((4, _B_PTS // _C, 8, _C), jnp.float32),
        mesh=mesh,
        scratch_types=scratch,
        compiler_params=pltpu.CompilerParams(
            needs_layout_passes=False, use_tc_tiling_on_sc=False),
    )


_sc_kernel = _make_kernel()


@jax.jit
def kernel(x, emb_0, emb_1, emb_2, emb_3, emb_4, emb_5, emb_6, emb_7,
           emb_8, emb_9, emb_10, emb_11, emb_12, emb_13, emb_14, emb_15):
    tables = [emb_0, emb_1, emb_2, emb_3, emb_4, emb_5, emb_6, emb_7,
              emb_8, emb_9, emb_10, emb_11, emb_12, emb_13, emb_14, emb_15]
    args = []
    for l in range(_N_LEVELS):
        tb = tables[l]
        if l in _RES_LEVELS:
            args.append(tb.reshape(-1))
        elif _DENSE[l]:
            r = _RES[l]
            args.append(jnp.concatenate(
                [tb, jnp.roll(tb, -1, axis=0),
                 jnp.roll(tb, -r, axis=0), jnp.roll(tb, -r - 1, axis=0)],
                axis=1))
        else:
            args.append(tb.reshape(_T // _C, _C, _NF)
                        .transpose(0, 2, 1).reshape(_T // 4, 8))
    xv = x.reshape(_B_PTS // _C, _C, 2).transpose(0, 2, 1).reshape(
        _B_PTS // _C, 2 * _C)
    out4 = _sc_kernel(xv, *args)
    return out4.transpose(1, 3, 0, 2).reshape(_B_PTS, 2 * _N_LEVELS)


# speedup vs baseline: 1.1983x; 1.0002x over previous
import jax
import jax.numpy as jnp
import numpy as np
from jax import lax
from jax.experimental import pallas as pl
from jax.experimental.pallas import tpu as pltpu
from jax.experimental.pallas import tpu_sc as plsc

_N_LEVELS = 16
_NF = 2
_LOG2_T = 19
_T = 2 ** _LOG2_T
_BASE_RES = 16
_FINEST_RES = 1024
_B_PTS = 1048576
_GROWTH = np.float32(
    np.exp((np.log(np.float32(_FINEST_RES)) - np.log(np.float32(_BASE_RES)))
           / (_N_LEVELS - 1)))
_PRIME1_I32 = int(np.uint32(2654435761).view(np.int32))
_HASH_MASK = _T - 1

_RES = []
_RESF = []
_GS = []
_TSIZE = []
_DENSE = []
for _i in range(_N_LEVELS):
    _resf = float(np.floor(np.float32(_BASE_RES) * _GROWTH ** np.float32(_i)))
    _r = int(_resf)
    _RES.append(_r)
    _RESF.append(np.float32(_resf))
    _GS.append(np.float32(1.0 / _resf))
    if _r * _r < _T:
        _TSIZE.append((_r + 1) ** 2)
        _DENSE.append(True)
    else:
        _TSIZE.append(_T)
        _DENSE.append(False)

_NC = 2
_NS = 16
_NW = _NC * _NS
_PW = _B_PTS // _NW
_C = 128
_VPC = _C // 16
_NCHUNK = _PW // _C

_RES_LEVELS = list(range(0, 8))
_BIG_LEVELS = list(range(8, 16))
_NGATH = [1 if _DENSE[l] else 8 for l in _BIG_LEVELS]
_GW = 8
_GOFF = list(np.cumsum([0] + _NGATH))
_TOTG = _GOFF[-1]


def _coords(x0, x1, l):
    rf = _RESF[l]
    b0 = (x0 * rf).astype(jnp.int32)
    b1 = (x1 * rf).astype(jnp.int32)
    return b0, b1


def _weights(x0, x1, b0, b1, l):
    gs = _GS[l]
    rf = _RESF[l]
    w0 = (x0 - b0.astype(jnp.float32) * gs) * rf
    w1 = (x1 - b1.astype(jnp.float32) * gs) * rf
    return w0, w1


def _corner_indices(b0, b1, l):
    if _DENSE[l]:
        r = _RES[l]
        i00 = b0 * r + b1
        return i00, i00 + 1, i00 + r, i00 + r + 1
    h0 = b0 ^ (b1 * _PRIME1_I32)
    h1 = b0 ^ ((b1 + 1) * _PRIME1_I32)
    h2 = (b0 + 1) ^ (b1 * _PRIME1_I32)
    h3 = (b0 + 1) ^ ((b1 + 1) * _PRIME1_I32)
    return (h0 & _HASH_MASK, h1 & _HASH_MASK,
            h2 & _HASH_MASK, h3 & _HASH_MASK)


def _lerp_store(corner_vals, w0, w1, out_ref, prow, l):
    u0 = 1.0 - w0
    u1 = 1.0 - w1
    for f in range(_NF):
        e00, e01, e10, e11 = corner_vals[f]
        c0 = e00 * u1 + e01 * w1
        c1 = e10 * u1 + e11 * w1
        o = c0 * u0 + c1 * w0
        plsc.store_scatter(
            out_ref, [jnp.full((16,), 2 * l + f, jnp.int32), prow], o)


def _hash_rows(i):
    p0 = ((i >> 7) << 8) + (i & 127)
    return p0 >> 3


def _sc_body(x_hbm, *refs):
    tbl_hbm = refs[0:_N_LEVELS]
    out_hbm = refs[_N_LEVELS]
    s = refs[_N_LEVELS + 1:]
    nres = len(_RES_LEVELS)
    tbl_v = s[0:nres]
    p = nres
    xi_v = s[p:p + 2]; p += 2
    x0_v = s[p:p + 2]; p += 2
    x1_v = s[p:p + 2]; p += 2
    out_v = s[p:p + 2]; p += 2
    idx_v = [s[p:p + _TOTG], s[p + _TOTG:p + 2 * _TOTG]]; p += 2 * _TOTG
    rows_v = [s[p:p + _TOTG], s[p + _TOTG:p + 2 * _TOTG]]; p += 2 * _TOTG
    gsem = s[p:p + 2]; p += 2
    osem = s[p:p + 2]

    wid = lax.axis_index("s") * _NC + lax.axis_index("c")
    cbase = wid * _NCHUNK

    for li, l in enumerate(_RES_LEVELS):
        pltpu.sync_copy(tbl_hbm[l], tbl_v[li])

    lane = lax.iota(jnp.int32, 16)

    def fire(par, g):
        pltpu.sync_copy(x_hbm.at[g], xi_v[par])

        def deint_body(v, c):
            x0_v[par][pl.ds(v * 16, 16)] = xi_v[par][pl.ds(v * 16, 16)]
            x1_v[par][pl.ds(v * 16, 16)] = xi_v[par][pl.ds(128 + v * 16, 16)]
            return c

        lax.fori_loop(0, _VPC, deint_body, 0)

        def idx_body(v, c):
            x0 = x0_v[par][pl.ds(v * 16, 16)]
            x1 = x1_v[par][pl.ds(v * 16, 16)]
            for li, l in enumerate(_BIG_LEVELS):
                b0, b1 = _coords(x0, x1, l)
                i00, i01, i10, i11 = _corner_indices(b0, b1, l)
                if _DENSE[l]:
                    ivs = (i00,)
                else:
                    r00, r01 = _hash_rows(i00), _hash_rows(i01)
                    r10, r11 = _hash_rows(i10), _hash_rows(i11)
                    ivs = (r00, r01, r10, r11,
                           r00 + 16, r01 + 16, r10 + 16, r11 + 16)
                for ci, iv in enumerate(ivs):
                    idx_v[par][_GOFF[li] + ci][pl.ds(v * 16, 16)] = iv
            return c

        lax.fori_loop(0, _VPC, idx_body, 0)

        for li, l in enumerate(_BIG_LEVELS):
            for ci in range(_NGATH[li]):
                gi = _GOFF[li] + ci
                pltpu.async_copy(
                    tbl_hbm[l].at[idx_v[par][gi]], rows_v[par][gi],
                    gsem[par])

    def wait_gathers(par):
        for li, l in enumerate(_BIG_LEVELS):
            for ci in range(_NGATH[li]):
                gi = _GOFF[li] + ci
                pltpu.make_async_copy(
                    tbl_hbm[l].at[idx_v[par][gi]], rows_v[par][gi],
                    gsem[par]).wait()

    def combine(par, g):
        @pl.when(g - cbase >= 2)
        def _drain_out():
            for a in range(4):
                pltpu.make_async_copy(
                    out_v[par].at[pl.ds(8 * a, 8), :], out_hbm.at[a, g],
                    osem[par]).wait()

        def res_body(v, c):
            x0 = x0_v[par][pl.ds(v * 16, 16)]
            x1 = x1_v[par][pl.ds(v * 16, 16)]
            prow = v * 16 + lane
            for li, l in enumerate(_RES_LEVELS):
                b0, b1 = _coords(x0, x1, l)
                w0, w1 = _weights(x0, x1, b0, b1, l)
                i00, i01, i10, i11 = _corner_indices(b0, b1, l)
                vals = []
                for f in range(_NF):
                    vals.append(tuple(
                        plsc.load_gather(tbl_v[li], [2 * i + f])
                        for i in (i00, i01, i10, i11)))
                _lerp_store(vals, w0, w1, out_v[par], prow, l)
            return c

        lax.fori_loop(0, _VPC, res_body, 0)

        wait_gathers(par)

        def big_body(v, c):
            x0 = x0_v[par][pl.ds(v * 16, 16)]
            x1 = x1_v[par][pl.ds(v * 16, 16)]
            prow = v * 16 + lane
            lidx = prow
            for li, l in enumerate(_BIG_LEVELS):
                b0, b1 = _coords(x0, x1, l)
                w0, w1 = _weights(x0, x1, b0, b1, l)
                gi = _GOFF[li]
                vals = []
                if _DENSE[l]:
                    for f in range(_NF):
                        rv = rows_v[par]
                        qcols = [jnp.full((16,), 2 * k + f, jnp.int32)
                                 for k in range(4)]
                        e00 = plsc.load_gather(rv[gi], [lidx, qcols[0]])
                        e01 = plsc.load_gather(rv[gi], [lidx, qcols[1]])
                        e10 = plsc.load_gather(rv[gi], [lidx, qcols[2]])
                        e11 = plsc.load_gather(rv[gi], [lidx, qcols[3]])
                        vals.append((e00, e01, e10, e11))
                else:
                    i00, i01, i10, i11 = _corner_indices(b0, b1, l)
                    for f in range(_NF):
                        o = 4 * f
                        rv = rows_v[par]
                        e00 = plsc.load_gather(rv[gi + o + 0], [lidx, i00 & 7])
                        e01 = plsc.load_gather(rv[gi + o + 1], [lidx, i01 & 7])
                        e10 = plsc.load_gather(rv[gi + o + 2], [lidx, i10 & 7])
                        e11 = plsc.load_gather(rv[gi + o + 3], [lidx, i11 & 7])
                        vals.append((e00, e01, e10, e11))
                _lerp_store(vals, w0, w1, out_v[par], prow, l)
            return c

        lax.fori_loop(0, _VPC, big_body, 0)

        for a in range(4):
            pltpu.async_copy(out_v[par].at[pl.ds(8 * a, 8), :],
                             out_hbm.at[a, g], osem[par])

    fire(0, cbase)

    def body(i, carry):
        g0 = cbase + 2 * i
        fire(1, g0 + 1)
        combine(0, g0)

        @pl.when(i < _NCHUNK // 2 - 1)
        def _fire_next():
            fire(0, g0 + 2)

        combine(1, g0 + 1)
        return carry

    lax.fori_loop(0, _NCHUNK // 2, body, 0)

    for par in range(2):
        for a in range(4):
            pltpu.make_async_copy(
                out_v[par].at[pl.ds(8 * a, 8), :],
                out_hbm.at[a, cbase], osem[par]).wait()


def _make_kernel():
    scratch = []
    scratch += [pltpu.VMEM((_TSIZE[l] * _NF,), jnp.float32)
                for l in _RES_LEVELS]
    scratch += [pltpu.VMEM((2 * _C,), jnp.float32)] * 2
    scratch += [pltpu.VMEM((_C,), jnp.float32)] * 2
    scratch += [pltpu.VMEM((_C,), jnp.float32)] * 2
    scratch += [pltpu.VMEM((2 * _N_LEVELS, _C), jnp.float32)] * 2
    scratch += [pltpu.VMEM((_C,), jnp.int32) for _ in range(2 * _TOTG)]
    scratch += [pltpu.VMEM((_C, _GW), jnp.float32) for _ in range(2 * _TOTG)]
    scratch += [pltpu.SemaphoreType.DMA] * 4
    mesh = plsc.VectorSubcoreMesh(core_axis_name="c", subcore_axis_name="s")
    return pl.kernel(
        _sc_body,
        out_type=jax.ShapeDtypeStruct((4, _B_PTS // _C, 8, _C), jnp.float32),
        mesh=mesh,
        scratch_types=scratch,
        compiler_params=pltpu.CompilerParams(
            needs_layout_passes=False, use_tc_tiling_on_sc=False),
    )


_sc_kernel = _make_kernel()


@jax.jit
def kernel(x, emb_0, emb_1, emb_2, emb_3, emb_4, emb_5, emb_6, emb_7,
           emb_8, emb_9, emb_10, emb_11, emb_12, emb_13, emb_14, emb_15):
    tables = [emb_0, emb_1, emb_2, emb_3, emb_4, emb_5, emb_6, emb_7,
              emb_8, emb_9, emb_10, emb_11, emb_12, emb_13, emb_14, emb_15]
    args = []
    for l in range(_N_LEVELS):
        tb = tables[l]
        if l in _RES_LEVELS:
            args.append(tb.reshape(-1))
        elif _DENSE[l]:
            r = _RES[l]
            args.append(jnp.concatenate(
                [tb, jnp.roll(tb, -1, axis=0),
                 jnp.roll(tb, -r, axis=0), jnp.roll(tb, -r - 1, axis=0)],
                axis=1))
        else:
            args.append(tb.reshape(_T // _C, _C, _NF)
                        .transpose(0, 2, 1).reshape(_T // 4, 8))
    xv = x.reshape(_B_PTS // _C, _C, 2).transpose(0, 2, 1).reshape(
        _B_PTS // _C, 2 * _C)
    out4 = _sc_kernel(xv, *args)
    return out4.transpose(1, 3, 0, 2).reshape(_B_PTS, 2 * _N_LEVELS)
